# Initial kernel scaffold; baseline (speedup 1.0000x reference)
import jax
import jax.numpy as jnp
from jax.experimental import pallas as pl


def kernel(x, edge_index, BU_edge_index, rootindex, W1, b1, W2, b2, W3, b3, W4, b4, Wfc, bfc):
    raise NotImplementedError("write your pallas kernel here")


# speedup vs baseline: 13.0640x; 13.0640x over previous
import functools

import jax
import jax.numpy as jnp
from jax import lax
from jax.experimental import pallas as pl
from jax.experimental.pallas import tpu as pltpu
from jax.experimental.pallas import tpu_sc as plsc

N = 10000
NP = 10240
E = 320000
D = 128
B_ROOT = 128
NTILES = 16
K = 128
EP = 321536
EPT = EP // NTILES
NCH = EPT // K
ROWS_PT = NP // NTILES
DEGW = 16

_mesh = plsc.VectorSubcoreMesh(core_axis_name="c", subcore_axis_name="s")


def _sc_deg_body(dst_hbm, zeros16_hbm, out_hbm, acc, dstbuf, onesbuf):
    c = lax.axis_index("c")
    t = lax.axis_index("s")
    pltpu.sync_copy(zeros16_hbm.at[pl.ds(t * ROWS_PT, ROWS_PT)],
                    acc.at[pl.ds(t * ROWS_PT, ROWS_PT)])

    def fill(r, carry):
        onesbuf[r, :] = jnp.ones((DEGW,), jnp.float32)
        return carry
    lax.fori_loop(0, K, fill, 0)
    plsc.subcore_barrier()

    base = t * EPT

    def body(g, carry):
        pltpu.sync_copy(dst_hbm.at[c].at[pl.ds(base + g * K, K)], dstbuf)
        pltpu.sync_copy(onesbuf, acc.at[dstbuf], add=True)
        return carry
    lax.fori_loop(0, NCH, body, 0)
    plsc.subcore_barrier()

    pltpu.sync_copy(acc.at[pl.ds(t * ROWS_PT, ROWS_PT)],
                    out_hbm.at[c].at[pl.ds(t * ROWS_PT, ROWS_PT)])


_sc_deg = functools.partial(
    pl.kernel,
    out_type=jax.ShapeDtypeStruct((2, NP, DEGW), jnp.float32),
    mesh=_mesh,
    scratch_types=[
        pltpu.VMEM_SHARED((NP, DEGW), jnp.float32),
        pltpu.VMEM((K,), jnp.int32),
        pltpu.VMEM((K, DEGW), jnp.float32),
    ],
)(_sc_deg_body)


def _sc_spmm_body(hflat_hbm, src_hbm, dst_hbm, zeros_hbm, out_hbm,
                  acc, srcbuf, dstbuf, rowbuf):
    c = lax.axis_index("c")
    t = lax.axis_index("s")
    pltpu.sync_copy(zeros_hbm.at[pl.ds(t * ROWS_PT, ROWS_PT)],
                    acc.at[pl.ds(t * ROWS_PT, ROWS_PT)])
    plsc.subcore_barrier()

    base = t * EPT
    off = c * NP

    def body(g, carry):
        pltpu.sync_copy(src_hbm.at[c].at[pl.ds(base + g * K, K)], srcbuf)
        pltpu.sync_copy(dst_hbm.at[c].at[pl.ds(base + g * K, K)], dstbuf)

        def shift(j, carry2):
            srcbuf[pl.ds(j * 16, 16)] = srcbuf[pl.ds(j * 16, 16)] + off
            return carry2
        lax.fori_loop(0, K // 16, shift, 0)

        pltpu.sync_copy(hflat_hbm.at[srcbuf], rowbuf)
        pltpu.sync_copy(rowbuf, acc.at[dstbuf], add=True)
        return carry
    lax.fori_loop(0, NCH, body, 0)
    plsc.subcore_barrier()

    pltpu.sync_copy(acc.at[pl.ds(t * ROWS_PT, ROWS_PT)],
                    out_hbm.at[c].at[pl.ds(t * ROWS_PT, ROWS_PT)])


_sc_spmm = functools.partial(
    pl.kernel,
    out_type=jax.ShapeDtypeStruct((2, NP, D), jnp.float32),
    mesh=_mesh,
    scratch_types=[
        pltpu.VMEM_SHARED((NP, D), jnp.float32),
        pltpu.VMEM((K,), jnp.int32),
        pltpu.VMEM((K,), jnp.int32),
        pltpu.VMEM((K, D), jnp.float32),
    ],
)(_sc_spmm_body)


def _sc_root_body(hflat_hbm, root_hbm, out_hbm, idxbuf, rbuf):
    c = lax.axis_index("c")
    t = lax.axis_index("s")

    @pl.when(t == 0)
    def _():
        pltpu.sync_copy(root_hbm, idxbuf)

        def shift(j, carry):
            idxbuf[pl.ds(j * 16, 16)] = idxbuf[pl.ds(j * 16, 16)] + c * NP
            return carry
        lax.fori_loop(0, B_ROOT // 16, shift, 0)
        pltpu.sync_copy(hflat_hbm.at[idxbuf], rbuf)
        pltpu.sync_copy(rbuf, out_hbm.at[c])


BLK = 640
NB = NP // BLK


def _tc_prep_body(deg_ref, x_ref, w_ref, dinv_ref, h1_ref):
    deg = deg_ref[0, :, 0]
    dinv = lax.rsqrt(deg + 1.0)
    h = jnp.dot(x_ref[...], w_ref[0], preferred_element_type=jnp.float32)
    dinv_ref[0, :, 0] = dinv
    h1_ref[0] = h * dinv[:, None]


def _tc_mid_body(acc_ref, h1_ref, dinv_ref, b_ref, w_ref, h2_ref):
    dinv = dinv_ref[0]
    t = dinv * (acc_ref[0] + h1_ref[0]) + b_ref[0]
    h = jnp.where(t > 0, t, jnp.exp(t) - 1.0)
    h2_ref[0] = jnp.dot(h, w_ref[0], preferred_element_type=jnp.float32) * dinv


def _tc_post_body(acc_ref, h2_ref, dinv_ref, b_ref, out_ref):
    dinv = dinv_ref[0]
    t = dinv * (acc_ref[0] + h2_ref[0]) + b_ref[0]
    out_ref[0] = jnp.where(t > 0, t, jnp.exp(t) - 1.0)


def _tc_fc_body(r_ref, w_ref, b_ref, out_ref):
    cat = jnp.concatenate([r_ref[0], r_ref[1]], axis=1)
    o = jnp.dot(cat, w_ref[...], preferred_element_type=jnp.float32) + b_ref[...]
    m = jnp.max(o, axis=1, keepdims=True)
    lse = jnp.log(jnp.sum(jnp.exp(o - m), axis=1, keepdims=True)) + m
    out_ref[...] = o - lse


def _tc_prep(deg, x, wst):
    return pl.pallas_call(
        _tc_prep_body,
        grid=(2, NB),
        in_specs=[
            pl.BlockSpec((1, BLK, DEGW), lambda c, j: (c, j, 0)),
            pl.BlockSpec((BLK, D), lambda c, j: (j, 0)),
            pl.BlockSpec((1, D, D), lambda c, j: (c, 0, 0)),
        ],
        out_specs=[
            pl.BlockSpec((1, BLK, 1), lambda c, j: (c, j, 0)),
            pl.BlockSpec((1, BLK, D), lambda c, j: (c, j, 0)),
        ],
        out_shape=[
            jax.ShapeDtypeStruct((2, NP, 1), jnp.float32),
            jax.ShapeDtypeStruct((2, NP, D), jnp.float32),
        ],
    )(deg, x, wst)


def _tc_mid(acc, h1, dinv, bst, wst):
    return pl.pallas_call(
        _tc_mid_body,
        grid=(2, NB),
        in_specs=[
            pl.BlockSpec((1, BLK, D), lambda c, j: (c, j, 0)),
            pl.BlockSpec((1, BLK, D), lambda c, j: (c, j, 0)),
            pl.BlockSpec((1, BLK, 1), lambda c, j: (c, j, 0)),
            pl.BlockSpec((1, 1, D), lambda c, j: (c, 0, 0)),
            pl.BlockSpec((1, D, D), lambda c, j: (c, 0, 0)),
        ],
        out_specs=pl.BlockSpec((1, BLK, D), lambda c, j: (c, j, 0)),
        out_shape=jax.ShapeDtypeStruct((2, NP, D), jnp.float32),
    )(acc, h1, dinv, bst, wst)


def _tc_post(acc, h2, dinv, bst):
    return pl.pallas_call(
        _tc_post_body,
        grid=(2, NB),
        in_specs=[
            pl.BlockSpec((1, BLK, D), lambda c, j: (c, j, 0)),
            pl.BlockSpec((1, BLK, D), lambda c, j: (c, j, 0)),
            pl.BlockSpec((1, BLK, 1), lambda c, j: (c, j, 0)),
            pl.BlockSpec((1, 1, D), lambda c, j: (c, 0, 0)),
        ],
        out_specs=pl.BlockSpec((1, BLK, D), lambda c, j: (c, j, 0)),
        out_shape=jax.ShapeDtypeStruct((2, NP, D), jnp.float32),
    )(acc, h2, dinv, bst)


def _tc_fc(r, wfc_pad, bfc_pad):
    return pl.pallas_call(
        _tc_fc_body,
        out_shape=jax.ShapeDtypeStruct((B_ROOT, D), jnp.float32),
    )(r, wfc_pad, bfc_pad)


_sc_root = functools.partial(
    pl.kernel,
    out_type=jax.ShapeDtypeStruct((2, B_ROOT, D), jnp.float32),
    mesh=_mesh,
    scratch_types=[
        pltpu.VMEM((B_ROOT,), jnp.int32),
        pltpu.VMEM((B_ROOT, D), jnp.float32),
    ],
)(_sc_root_body)


def kernel(x, edge_index, BU_edge_index, rootindex,
           W1, b1, W2, b2, W3, b3, W4, b4, Wfc, bfc):
    pad = jnp.full((2, EP - E), N, jnp.int32)
    src_all = jnp.concatenate(
        [jnp.stack([edge_index[0], BU_edge_index[0]]), pad], axis=1)
    dst_all = jnp.concatenate(
        [jnp.stack([edge_index[1], BU_edge_index[1]]), pad], axis=1)
    wst1 = jnp.stack([W1, W3])
    wst2 = jnp.stack([W2, W4])
    bst1 = jnp.stack([b1, b3]).reshape(2, 1, D)
    bst2 = jnp.stack([b2, b4]).reshape(2, 1, D)
    x_pad = jnp.concatenate([x, jnp.zeros((NP - N, D), jnp.float32)])
    zeros = jnp.zeros((NP, D), jnp.float32)
    zeros16 = jnp.zeros((NP, DEGW), jnp.float32)
    wfc_pad = jnp.zeros((2 * D, D), jnp.float32).at[:, :4].set(Wfc)
    bfc_pad = jnp.full((D,), -1e30, jnp.float32).at[:4].set(bfc)

    deg = _sc_deg(dst_all, zeros16)
    dinv, h1 = _tc_prep(deg, x_pad, wst1)
    acc1 = _sc_spmm(h1.reshape(2 * NP, D), src_all, dst_all, zeros)
    h2 = _tc_mid(acc1, h1, dinv, bst1, wst2)
    acc2 = _sc_spmm(h2.reshape(2 * NP, D), src_all, dst_all, zeros)
    full = _tc_post(acc2, h2, dinv, bst2)
    roots = _sc_root(full.reshape(2 * NP, D), rootindex)
    out = _tc_fc(roots, wfc_pad, bfc_pad)
    return out[:, :4]
